# baseline (speedup 1.0000x reference)
import functools

import jax
import jax.numpy as jnp
from jax import lax
from jax.experimental import pallas as pl
from jax.experimental.pallas import tpu as pltpu


def _encode_kernel(x_ref, w1_ref, b1_ref, w2_ref, b2_ref, emb_ref):
    pooled = jnp.sum(x_ref[...], axis=1)
    h = jnp.dot(pooled, w1_ref[...], preferred_element_type=jnp.float32) + b1_ref[...]
    h = jnp.maximum(h, 0.0)
    emb_ref[...] = jnp.dot(h, w2_ref[...], preferred_element_type=jnp.float32) + b2_ref[...]


def _encode(x, w1s, b1, w2, b2, tb):
    B, T, D = x.shape
    H = w1s.shape[1]
    return pl.pallas_call(
        _encode_kernel,
        out_shape=jax.ShapeDtypeStruct((B, H), jnp.float32),
        grid=(pl.cdiv(B, tb),),
        in_specs=[
            pl.BlockSpec((tb, T, D), lambda i: (i, 0, 0)),
            pl.BlockSpec((D, H), lambda i: (0, 0)),
            pl.BlockSpec((1, H), lambda i: (0, 0)),
            pl.BlockSpec((H, H), lambda i: (0, 0)),
            pl.BlockSpec((1, H), lambda i: (0, 0)),
        ],
        out_specs=pl.BlockSpec((tb, H), lambda i: (i, 0)),
        compiler_params=pltpu.CompilerParams(
            dimension_semantics=("parallel",)),
    )(x, w1s, b1, w2, b2)


def _loss_kernel(tr: int, erow_ref, eall_ref, lbl_ref, out_ref):
    i = pl.program_id(0)
    er = erow_ref[...]
    ea = eall_ref[...]
    B = ea.shape[0]

    sq_r = jnp.sum(er * er, axis=1, keepdims=True)
    sq_a = jnp.sum(ea * ea, axis=1)[None, :]
    gram = lax.dot_general(
        er, ea, dimension_numbers=(((1,), (1,)), ((), ())),
        preferred_element_type=jnp.float32)
    d2 = jnp.maximum(sq_r + sq_a - 2.0 * gram, 0.0)

    row = i * tr + lax.broadcasted_iota(jnp.int32, (tr, B), 0)
    col = lax.broadcasted_iota(jnp.int32, (tr, B), 1)
    mask = col > row

    dist = jnp.sqrt(jnp.where(mask, d2, 1.0))
    diff = dist - lbl_ref[...]
    se = jnp.where(mask, diff * diff, 0.0)
    out_ref[0, 0, 0] = jnp.sum(se)


def _loss_partials(emb, labels, tr):
    B, H = emb.shape
    g = pl.cdiv(B, tr)
    return pl.pallas_call(
        functools.partial(_loss_kernel, tr),
        out_shape=jax.ShapeDtypeStruct((g, 1, 1), jnp.float32),
        grid=(g,),
        in_specs=[
            pl.BlockSpec((tr, H), lambda i: (i, 0)),
            pl.BlockSpec((B, H), lambda i: (0, 0)),
            pl.BlockSpec((tr, B), lambda i: (i, 0)),
        ],
        out_specs=pl.BlockSpec((1, 1, 1), lambda i: (i, 0, 0),
                               memory_space=pltpu.MemorySpace.SMEM),
        compiler_params=pltpu.CompilerParams(
            dimension_semantics=("parallel",)),
    )(emb, emb, labels)


def kernel(x, labels, w1, b1, w2, b2):
    B, T, D = x.shape
    w1s = w1 * (1.0 / T)
    emb = _encode(x, w1s, b1, w2, b2, tb=128)
    partials = _loss_partials(emb, labels, tr=128)
    num_active = B * (B - 1) // 2
    loss = jnp.sum(partials) * (1.0 / jnp.float32(num_active))
    return emb, loss


# speedup vs baseline: 1.2895x; 1.2895x over previous
import functools

import jax
import jax.numpy as jnp
from jax import lax
from jax.experimental import pallas as pl
from jax.experimental.pallas import tpu as pltpu


def _fused_kernel(G1, NP, tb, tr, inv_T, inv_active,
                  xa, ia, ja, ka, pa,
                  x_ref, lbl_ref, w1_ref, b1_ref, w2_ref, b2_ref,
                  emb_ref, loss_ref, escr_ref, acc_ref):
    s = pl.program_id(0)

    @pl.when(ka[s] == 0)
    def _encode_step():
        pooled = jnp.sum(x_ref[...], axis=1) * inv_T
        h = jnp.dot(pooled, w1_ref[...], preferred_element_type=jnp.float32) + b1_ref[...]
        h = jnp.maximum(h, 0.0)
        e = jnp.dot(h, w2_ref[...], preferred_element_type=jnp.float32) + b2_ref[...]
        emb_ref[...] = e
        escr_ref[pl.ds(xa[s] * tb, tb), :] = e

    @pl.when(ka[s] == 1)
    def _pair_step():
        i = ia[s]
        j = ja[s]
        er = escr_ref[pl.ds(i * tr, tr), :]
        ec = escr_ref[pl.ds(j * tr, tr), :]
        sq_r = jnp.sum(er * er, axis=1, keepdims=True)
        sq_c = jnp.sum(ec * ec, axis=1)[None, :]
        gram = lax.dot_general(
            er, ec, dimension_numbers=(((1,), (1,)), ((), ())),
            preferred_element_type=jnp.float32)
        d2 = jnp.maximum(sq_r + sq_c - 2.0 * gram, 0.0)

        row = i * tr + lax.broadcasted_iota(jnp.int32, (tr, tr), 0)
        col = j * tr + lax.broadcasted_iota(jnp.int32, (tr, tr), 1)
        mask = col > row

        dist = jnp.sqrt(jnp.where(mask, d2, 1.0))
        diff = dist - lbl_ref[...]
        se = jnp.where(mask, diff * diff, 0.0)
        part = jnp.sum(se, axis=0, keepdims=True)

        p = pa[s]
        prev = jnp.where(p == 0, jnp.zeros_like(acc_ref[...]), acc_ref[...])
        new = prev + part
        acc_ref[...] = new

        @pl.when(p == NP - 1)
        def _finish():
            loss_ref[0, 0] = jnp.sum(new) * inv_active


def _build_schedule(G1, G2, r):
    pairs = sorted(((i, j) for i in range(G2) for j in range(i, G2)),
                   key=lambda p: (max(p), p[0]))
    steps = []
    enc_done = 0
    pending = list(pairs)
    while enc_done < G1 or pending:
        if pending and (max(pending[0]) + 1) * r <= enc_done:
            steps.append(("p", pending.pop(0)))
        elif enc_done < G1:
            steps.append(("e", enc_done))
            enc_done += 1
        else:
            steps.append(("p", pending.pop(0)))

    S = len(steps)
    xa = [0] * S
    ia = [0] * S
    ja = [0] * S
    ka = [0] * S
    pa = [0] * S
    last_x = 0
    p_ord = 0
    for s, (kind, v) in enumerate(steps):
        if kind == "e":
            last_x = v
            xa[s] = v
        else:
            xa[s] = last_x
            ka[s] = 1
            ia[s], ja[s] = v
            pa[s] = p_ord
            p_ord += 1
    nxt = None
    for s in range(S - 1, -1, -1):
        if ka[s] == 1:
            nxt = (ia[s], ja[s])
        elif nxt is not None:
            ia[s], ja[s] = nxt
    return xa, ia, ja, ka, pa, len(pairs)


def _sts_fused(x, labels, w1, b1, w2, b2, tb, tr):
    B, T, D = x.shape
    H = w1.shape[1]
    G1 = B // tb
    G2 = B // tr
    r = tr // tb
    xa, ia, ja, ka, pa, NP = _build_schedule(G1, G2, r)
    S = len(xa)
    arrs = [jnp.asarray(a, dtype=jnp.int32) for a in (xa, ia, ja, ka, pa)]
    num_active = B * (B - 1) // 2

    body = functools.partial(
        _fused_kernel, G1, NP, tb, tr,
        float(1.0 / T), float(1.0 / num_active))

    emb, loss = pl.pallas_call(
        body,
        out_shape=(jax.ShapeDtypeStruct((B, H), jnp.float32),
                   jax.ShapeDtypeStruct((1, 1), jnp.float32)),
        grid_spec=pltpu.PrefetchScalarGridSpec(
            num_scalar_prefetch=5,
            grid=(S,),
            in_specs=[
                pl.BlockSpec((tb, T, D), lambda s, xa, ia, ja, ka, pa: (xa[s], 0, 0)),
                pl.BlockSpec((tr, tr), lambda s, xa, ia, ja, ka, pa: (ia[s], ja[s])),
                pl.BlockSpec((D, H), lambda s, *_: (0, 0)),
                pl.BlockSpec((1, H), lambda s, *_: (0, 0)),
                pl.BlockSpec((H, H), lambda s, *_: (0, 0)),
                pl.BlockSpec((1, H), lambda s, *_: (0, 0)),
            ],
            out_specs=(
                pl.BlockSpec((tb, H), lambda s, xa, ia, ja, ka, pa: (xa[s], 0)),
                pl.BlockSpec((1, 1), lambda s, *_: (0, 0),
                             memory_space=pltpu.MemorySpace.SMEM),
            ),
            scratch_shapes=[
                pltpu.VMEM((B, H), jnp.float32),
                pltpu.VMEM((1, tr), jnp.float32),
            ],
        ),
        compiler_params=pltpu.CompilerParams(
            dimension_semantics=("arbitrary",)),
    )(*arrs, x, labels, w1, b1, w2, b2)
    return emb, loss[0, 0]


def kernel(x, labels, w1, b1, w2, b2):
    return _sts_fused(x, labels, w1, b1, w2, b2, tb=128, tr=256)


# speedup vs baseline: 1.8026x; 1.3979x over previous
import jax
import jax.numpy as jnp
from jax import lax
from jax.experimental import pallas as pl
from jax.experimental.pallas import tpu as pltpu

_TB = 128
_DEPTH = 2


def _sts_kernel(x_hbm, lbl_hbm, w1_ref, b1_ref, w2_ref, b2_ref,
                emb_ref, loss_ref,
                xbuf, lblbuf, pooled, xsem, lsem):
    B, T, D = x_hbm.shape
    n_chunks = B // _TB
    inv_t = 1.0 / T

    pltpu.make_async_copy(lbl_hbm, lblbuf, lsem).start()

    def start(c):
        pltpu.make_async_copy(
            x_hbm.at[pl.ds(c * _TB, _TB)], xbuf.at[c % _DEPTH],
            xsem.at[c % _DEPTH]).start()

    def wait(c):
        s = c % _DEPTH
        pltpu.make_async_copy(xbuf.at[s], xbuf.at[s], xsem.at[s]).wait()

    start(0)
    for c in range(n_chunks):
        if c + 1 < n_chunks:
            start(c + 1)
        wait(c)
        pooled[pl.ds(c * _TB, _TB), :] = jnp.sum(xbuf[c % _DEPTH], axis=1) * inv_t

    h = jnp.dot(pooled[...], w1_ref[...], preferred_element_type=jnp.float32)
    h = jnp.maximum(h + b1_ref[...], 0.0)
    e = jnp.dot(h, w2_ref[...], preferred_element_type=jnp.float32) + b2_ref[...]
    emb_ref[...] = e

    sq = jnp.sum(e * e, axis=1, keepdims=True)
    gram = lax.dot_general(
        e, e, dimension_numbers=(((1,), (1,)), ((), ())),
        preferred_element_type=jnp.float32)
    d2 = jnp.maximum(sq + jnp.transpose(sq) - 2.0 * gram, 0.0)

    row = lax.broadcasted_iota(jnp.int32, (B, B), 0)
    col = lax.broadcasted_iota(jnp.int32, (B, B), 1)
    mask = col > row
    dist = jnp.sqrt(jnp.where(mask, d2, 1.0))

    pltpu.make_async_copy(lblbuf, lblbuf, lsem).wait()
    diff = dist - lblbuf[...]
    se = jnp.where(mask, diff * diff, 0.0)
    inv_active = 1.0 / float(B * (B - 1) // 2)
    loss_ref[0, 0] = jnp.sum(se) * inv_active


def kernel(x, labels, w1, b1, w2, b2):
    B, T, D = x.shape
    H = w1.shape[1]
    anyspec = pl.BlockSpec(memory_space=pl.ANY)
    vmem = pl.BlockSpec(memory_space=pltpu.MemorySpace.VMEM)
    smem = pl.BlockSpec(memory_space=pltpu.MemorySpace.SMEM)
    emb, loss = pl.pallas_call(
        _sts_kernel,
        out_shape=(jax.ShapeDtypeStruct((B, H), jnp.float32),
                   jax.ShapeDtypeStruct((1, 1), jnp.float32)),
        in_specs=[anyspec, anyspec, vmem, vmem, vmem, vmem],
        out_specs=(vmem, smem),
        scratch_shapes=[
            pltpu.VMEM((_DEPTH, _TB, T, D), jnp.float32),
            pltpu.VMEM((B, B), jnp.float32),
            pltpu.VMEM((B, D), jnp.float32),
            pltpu.SemaphoreType.DMA((_DEPTH,)),
            pltpu.SemaphoreType.DMA,
        ],
    )(x, labels, w1, b1, w2, b2)
    return emb, loss[0, 0]


# speedup vs baseline: 1.9415x; 1.0770x over previous
import jax
import jax.numpy as jnp
from jax import lax
from jax.experimental import pallas as pl
from jax.experimental.pallas import tpu as pltpu

_NX = 4
_NT = 4


def _sts_kernel(x_hbm, lbl_hbm, w1_ref, b1_ref, w2_ref, b2_ref,
                emb_ref, loss_ref,
                xs, lbls, xsem, lsem):
    B, T, D = x_hbm.shape
    tbx = B // _NX
    tt = B // _NT
    inv_t = 1.0 / T
    pairs = [(i, j) for i in range(_NT) for j in range(i, _NT)]

    for c in range(_NX):
        pltpu.make_async_copy(
            x_hbm.at[pl.ds(c * tbx, tbx)], xs.at[pl.ds(c * tbx, tbx)],
            xsem.at[c]).start()
    for t, (i, j) in enumerate(pairs):
        pltpu.make_async_copy(
            lbl_hbm.at[pl.ds(i * tt, tt), pl.ds(j * tt, tt)], lbls.at[t],
            lsem.at[t]).start()

    pooled_parts = []
    for c in range(_NX):
        dst = xs.at[pl.ds(c * tbx, tbx)]
        pltpu.make_async_copy(dst, dst, xsem.at[c]).wait()
        pooled_parts.append(jnp.sum(xs[pl.ds(c * tbx, tbx)], axis=1) * inv_t)
    pooled = jnp.concatenate(pooled_parts, axis=0)

    h = jnp.dot(pooled, w1_ref[...], preferred_element_type=jnp.float32)
    h = jnp.maximum(h + b1_ref[...], 0.0)
    e = jnp.dot(h, w2_ref[...], preferred_element_type=jnp.float32) + b2_ref[...]
    emb_ref[...] = e

    sq = jnp.sum(e * e, axis=1, keepdims=True)

    acc = jnp.zeros((1, tt), dtype=jnp.float32)
    for t, (i, j) in enumerate(pairs):
        ei = e[i * tt:(i + 1) * tt]
        ej = e[j * tt:(j + 1) * tt]
        gram = lax.dot_general(
            ei, ej, dimension_numbers=(((1,), (1,)), ((), ())),
            preferred_element_type=jnp.float32)
        d2 = jnp.maximum(
            sq[i * tt:(i + 1) * tt]
            + jnp.transpose(sq[j * tt:(j + 1) * tt]) - 2.0 * gram, 0.0)
        pltpu.make_async_copy(lbls.at[t], lbls.at[t], lsem.at[t]).wait()
        lbl_t = lbls[t]
        if i == j:
            row = lax.broadcasted_iota(jnp.int32, (tt, tt), 0)
            col = lax.broadcasted_iota(jnp.int32, (tt, tt), 1)
            mask = col > row
            diff = jnp.sqrt(jnp.where(mask, d2, 1.0)) - lbl_t
            se = jnp.where(mask, diff * diff, 0.0)
        else:
            diff = jnp.sqrt(d2) - lbl_t
            se = diff * diff
        acc = acc + jnp.sum(se, axis=0, keepdims=True)

    inv_active = 1.0 / float(B * (B - 1) // 2)
    loss_ref[0, 0] = jnp.sum(acc) * inv_active


def kernel(x, labels, w1, b1, w2, b2):
    B, T, D = x.shape
    H = w1.shape[1]
    tt = B // _NT
    np_pairs = _NT * (_NT + 1) // 2
    anyspec = pl.BlockSpec(memory_space=pl.ANY)
    vmem = pl.BlockSpec(memory_space=pltpu.MemorySpace.VMEM)
    smem = pl.BlockSpec(memory_space=pltpu.MemorySpace.SMEM)
    emb, loss = pl.pallas_call(
        _sts_kernel,
        out_shape=(jax.ShapeDtypeStruct((B, H), jnp.float32),
                   jax.ShapeDtypeStruct((1, 1), jnp.float32)),
        in_specs=[anyspec, anyspec, vmem, vmem, vmem, vmem],
        out_specs=(vmem, smem),
        scratch_shapes=[
            pltpu.VMEM((B, T, D), jnp.float32),
            pltpu.VMEM((np_pairs, tt, tt), jnp.float32),
            pltpu.SemaphoreType.DMA((_NX,)),
            pltpu.SemaphoreType.DMA((np_pairs,)),
        ],
    )(x, labels, w1, b1, w2, b2)
    return emb, loss[0, 0]


# speedup vs baseline: 2.1552x; 1.1101x over previous
import jax
import jax.numpy as jnp
from jax import lax
from jax.experimental import pallas as pl
from jax.experimental.pallas import tpu as pltpu

_NT = 4


def _sts_kernel(x_hbm, lbl_hbm, w1_ref, b1_ref, w2_ref, b2_ref,
                emb_ref, loss_ref,
                xs, lbls, xsem, lsem):
    B, T, D = x_hbm.shape
    tt = B // _NT
    inv_t = 1.0 / T
    pairs = [(i, c) for c in range(_NT) for i in range(c + 1)]
    pidx = {p: t for t, p in enumerate(pairs)}

    for c in range(_NT):
        pltpu.make_async_copy(
            x_hbm.at[pl.ds(c * tt, tt)], xs.at[c], xsem.at[c]).start()
        for i in range(c + 1):
            t = pidx[(i, c)]
            pltpu.make_async_copy(
                lbl_hbm.at[pl.ds(i * tt, tt), pl.ds(c * tt, tt)], lbls.at[t],
                lsem.at[t]).start()

    w1 = w1_ref[...]
    b1 = b1_ref[...]
    w2 = w2_ref[...]
    b2 = b2_ref[...]

    es = []
    sqs = []
    acc = jnp.zeros((1, tt), dtype=jnp.float32)
    for c in range(_NT):
        pltpu.make_async_copy(xs.at[c], xs.at[c], xsem.at[c]).wait()
        pooled = jnp.sum(xs[c], axis=1) * inv_t
        h = jnp.maximum(
            jnp.dot(pooled, w1, preferred_element_type=jnp.float32) + b1, 0.0)
        e = jnp.dot(h, w2, preferred_element_type=jnp.float32) + b2
        emb_ref[pl.ds(c * tt, tt), :] = e
        es.append(e)
        sqs.append(jnp.sum(e * e, axis=1, keepdims=True))

        for i in range(c + 1):
            t = pidx[(i, c)]
            gram = lax.dot_general(
                es[i], e, dimension_numbers=(((1,), (1,)), ((), ())),
                preferred_element_type=jnp.float32)
            d2 = jnp.maximum(
                sqs[i] + jnp.transpose(sqs[c]) - 2.0 * gram, 0.0)
            pltpu.make_async_copy(lbls.at[t], lbls.at[t], lsem.at[t]).wait()
            diff = jnp.sqrt(d2) - lbls[t]
            if i == c:
                row = lax.broadcasted_iota(jnp.int32, (tt, tt), 0)
                col = lax.broadcasted_iota(jnp.int32, (tt, tt), 1)
                se = jnp.where(col > row, diff * diff, 0.0)
            else:
                se = diff * diff
            acc = acc + jnp.sum(se, axis=0, keepdims=True)

    inv_active = 1.0 / float(B * (B - 1) // 2)
    loss_ref[0, 0] = jnp.sum(acc) * inv_active


def kernel(x, labels, w1, b1, w2, b2):
    B, T, D = x.shape
    H = w1.shape[1]
    tt = B // _NT
    np_pairs = _NT * (_NT + 1) // 2
    anyspec = pl.BlockSpec(memory_space=pl.ANY)
    vmem = pl.BlockSpec(memory_space=pltpu.MemorySpace.VMEM)
    smem = pl.BlockSpec(memory_space=pltpu.MemorySpace.SMEM)
    emb, loss = pl.pallas_call(
        _sts_kernel,
        out_shape=(jax.ShapeDtypeStruct((B, H), jnp.float32),
                   jax.ShapeDtypeStruct((1, 1), jnp.float32)),
        in_specs=[anyspec, anyspec, vmem, vmem, vmem, vmem],
        out_specs=(vmem, smem),
        scratch_shapes=[
            pltpu.VMEM((_NT, tt, T, D), jnp.float32),
            pltpu.VMEM((np_pairs, tt, tt), jnp.float32),
            pltpu.SemaphoreType.DMA((_NT,)),
            pltpu.SemaphoreType.DMA((np_pairs,)),
        ],
    )(x, labels, w1, b1, w2, b2)
    return emb, loss[0, 0]


# speedup vs baseline: 2.3515x; 1.0911x over previous
import jax
import jax.numpy as jnp
from jax import lax
from jax.experimental import pallas as pl
from jax.experimental.pallas import tpu as pltpu

_NT = 4


def _sts_kernel(x_hbm, lbl_hbm, w1_ref, b1_ref, w2_ref, b2_ref,
                emb_ref, loss_ref,
                xs, lbls, xsem, lsem):
    B, T, D = x_hbm.shape
    tt = B // _NT
    inv_t = 1.0 / T
    for c in range(_NT):
        pltpu.make_async_copy(
            x_hbm.at[pl.ds(c * tt, tt)], xs.at[c], xsem.at[c]).start()
        n = (c + 1) * tt
        pltpu.make_async_copy(
            lbl_hbm.at[pl.ds(0, n), pl.ds(c * tt, tt)],
            lbls.at[pl.ds(0, n), pl.ds(c * tt, tt)], lsem.at[c]).start()

    w1 = w1_ref[...]
    b1 = b1_ref[...]
    w2 = w2_ref[...]
    b2 = b2_ref[...]

    es = []
    sqs = []
    acc = jnp.zeros((1, tt), dtype=jnp.float32)
    for c in range(_NT):
        pltpu.make_async_copy(xs.at[c], xs.at[c], xsem.at[c]).wait()
        pooled = jnp.sum(xs[c], axis=1) * inv_t
        h = jnp.maximum(
            jnp.dot(pooled, w1, preferred_element_type=jnp.float32) + b1, 0.0)
        e = jnp.dot(h, w2, preferred_element_type=jnp.float32) + b2
        emb_ref[pl.ds(c * tt, tt), :] = e
        es.append(e)
        sqs.append(jnp.sum(e * e, axis=1, keepdims=True))

        pltpu.make_async_copy(
            lbls.at[pl.ds(0, (c + 1) * tt), pl.ds(c * tt, tt)],
            lbls.at[pl.ds(0, (c + 1) * tt), pl.ds(c * tt, tt)],
            lsem.at[c]).wait()
        sq_col = jnp.transpose(sqs[c])
        for i in range(c + 1):
            gram = lax.dot_general(
                es[i], e, dimension_numbers=(((1,), (1,)), ((), ())),
                preferred_element_type=jnp.float32)
            d2 = jnp.maximum(sqs[i] + sq_col - 2.0 * gram, 0.0)
            diff = jnp.sqrt(d2) - lbls[pl.ds(i * tt, tt), pl.ds(c * tt, tt)]
            if i == c:
                row = lax.broadcasted_iota(jnp.int32, (tt, tt), 0)
                col = lax.broadcasted_iota(jnp.int32, (tt, tt), 1)
                se = jnp.where(col > row, diff * diff, 0.0)
            else:
                se = diff * diff
            acc = acc + jnp.sum(se, axis=0, keepdims=True)

    inv_active = 1.0 / float(B * (B - 1) // 2)
    loss_ref[0, 0] = jnp.sum(acc) * inv_active


def kernel(x, labels, w1, b1, w2, b2):
    B, T, D = x.shape
    H = w1.shape[1]
    tt = B // _NT
    anyspec = pl.BlockSpec(memory_space=pl.ANY)
    vmem = pl.BlockSpec(memory_space=pltpu.MemorySpace.VMEM)
    smem = pl.BlockSpec(memory_space=pltpu.MemorySpace.SMEM)
    emb, loss = pl.pallas_call(
        _sts_kernel,
        out_shape=(jax.ShapeDtypeStruct((B, H), jnp.float32),
                   jax.ShapeDtypeStruct((1, 1), jnp.float32)),
        in_specs=[anyspec, anyspec, vmem, vmem, vmem, vmem],
        out_specs=(vmem, smem),
        scratch_shapes=[
            pltpu.VMEM((_NT, tt, T, D), jnp.float32),
            pltpu.VMEM((B, B), jnp.float32),
            pltpu.SemaphoreType.DMA((_NT,)),
            pltpu.SemaphoreType.DMA((_NT,)),
        ],
    )(x, labels, w1, b1, w2, b2)
    return emb, loss[0, 0]
